# Initial kernel scaffold; baseline (speedup 1.0000x reference)
import jax
import jax.numpy as jnp
from jax.experimental import pallas as pl


def kernel(inputs, codebook):
    raise NotImplementedError("write your pallas kernel here")


# speedup vs baseline: 3.8455x; 3.8455x over previous
import functools

import jax
import jax.numpy as jnp
from jax.experimental import pallas as pl
from jax.experimental.pallas import tpu as pltpu

_NUM_EMBEDDINGS = 8192
_EMBEDDING_DIM = 32
_COMMITMENT_COST = 0.25

_TM = 256


def _vq_kernel(x_ref, cb_ref, q_ref, loss_ref, perp_ref, counts_ref, loss_acc_ref):
    i = pl.program_id(0)
    nblocks = pl.num_programs(0)
    n_tok = nblocks * _TM

    x = x_ref[...]
    cb = cb_ref[...]

    x2 = jnp.sum(x * x, axis=1, keepdims=True)
    c2 = jnp.sum(cb * cb, axis=1)[None, :]
    dots = jax.lax.dot_general(
        x, cb,
        dimension_numbers=(((1,), (1,)), ((), ())),
        precision=jax.lax.Precision.DEFAULT,
        preferred_element_type=jnp.float32,
    )
    dist = x2 + c2 - 2.0 * dots

    m = jnp.min(dist, axis=1, keepdims=True)
    iota = jax.lax.broadcasted_iota(jnp.int32, dist.shape, 1)
    cand = jnp.where(dist == m, iota, _NUM_EMBEDDINGS)
    a = jnp.min(cand, axis=1, keepdims=True)
    oh = (iota == a).astype(jnp.float32)

    q = jax.lax.dot_general(
        oh, cb,
        dimension_numbers=(((1,), (0,)), ((), ())),
        precision=jax.lax.Precision.HIGHEST,
        preferred_element_type=jnp.float32,
    )
    q_ref[...] = q

    @pl.when(i == 0)
    def _init():
        counts_ref[...] = jnp.zeros_like(counts_ref)
        loss_acc_ref[0, 0] = 0.0

    diff = q - x
    loss_acc_ref[0, 0] += jnp.sum(diff * diff)
    counts_ref[...] += jnp.sum(oh, axis=0, keepdims=True)

    @pl.when(i == nblocks - 1)
    def _finalize():
        loss_ref[0, 0] = (1.0 + _COMMITMENT_COST) * loss_acc_ref[0, 0] / (
            n_tok * _EMBEDDING_DIM)
        p = counts_ref[...] / n_tok
        ent = -jnp.sum(p * jnp.log(p + 1e-10))
        perp_ref[0, 0] = jnp.exp(ent)


@functools.partial(jax.jit, static_argnames=())
def _vq(flat_x, cb):
    n_tok = flat_x.shape[0]
    nblocks = n_tok // _TM
    q, loss, perp = pl.pallas_call(
        _vq_kernel,
        grid=(nblocks,),
        in_specs=[
            pl.BlockSpec((_TM, _EMBEDDING_DIM), lambda i: (i, 0)),
            pl.BlockSpec((_NUM_EMBEDDINGS, _EMBEDDING_DIM), lambda i: (0, 0)),
        ],
        out_specs=[
            pl.BlockSpec((_TM, _EMBEDDING_DIM), lambda i: (i, 0)),
            pl.BlockSpec((1, 1), lambda i: (0, 0), memory_space=pltpu.SMEM),
            pl.BlockSpec((1, 1), lambda i: (0, 0), memory_space=pltpu.SMEM),
        ],
        out_shape=[
            jax.ShapeDtypeStruct((n_tok, _EMBEDDING_DIM), jnp.float32),
            jax.ShapeDtypeStruct((1, 1), jnp.float32),
            jax.ShapeDtypeStruct((1, 1), jnp.float32),
        ],
        scratch_shapes=[
            pltpu.VMEM((1, _NUM_EMBEDDINGS), jnp.float32),
            pltpu.SMEM((1, 1), jnp.float32),
        ],
    )(flat_x, cb)
    return q, loss[0, 0], perp[0, 0]


def kernel(inputs, codebook):
    input_shape = inputs.shape
    flat_x = inputs.reshape(-1, _EMBEDDING_DIM).astype(jnp.float32)
    cb = codebook.astype(jnp.float32)
    q, loss, perp = _vq(flat_x, cb)
    quantized_ste = q.reshape(input_shape).astype(inputs.dtype)
    return (quantized_ste, loss, perp)


# speedup vs baseline: 6.6791x; 1.7369x over previous
import functools

import jax
import jax.numpy as jnp
from jax.experimental import pallas as pl
from jax.experimental.pallas import tpu as pltpu

_NUM_EMBEDDINGS = 8192
_EMBEDDING_DIM = 32
_COMMITMENT_COST = 0.25

_TM = 256


def _vq_kernel(x_ref, cb_ref, q_ref, loss_ref, perp_ref, counts_ref, loss_acc_ref):
    i = pl.program_id(0)
    nblocks = pl.num_programs(0)
    n_tok = nblocks * _TM

    x = x_ref[...]
    cb = cb_ref[...]

    x2 = jnp.sum(x * x, axis=1, keepdims=True)
    c2 = jnp.sum(cb * cb, axis=1)[None, :]
    dots = jax.lax.dot_general(
        x, cb,
        dimension_numbers=(((1,), (1,)), ((), ())),
        precision=jax.lax.Precision.DEFAULT,
        preferred_element_type=jnp.float32,
    )
    dist = x2 + c2 - 2.0 * dots

    m = jnp.min(dist, axis=1, keepdims=True)
    iota = jax.lax.broadcasted_iota(jnp.int32, dist.shape, 1)
    cand = jnp.where(dist == m, iota, _NUM_EMBEDDINGS)
    a = jnp.min(cand, axis=1, keepdims=True)
    oh = (iota == a).astype(jnp.float32)

    q = jax.lax.dot_general(
        oh, cb,
        dimension_numbers=(((1,), (0,)), ((), ())),
        precision=jax.lax.Precision.DEFAULT,
        preferred_element_type=jnp.float32,
    )
    q_ref[...] = q

    @pl.when(i == 0)
    def _init():
        counts_ref[...] = jnp.zeros_like(counts_ref)
        loss_acc_ref[0, 0] = 0.0

    diff = q - x
    loss_acc_ref[0, 0] += jnp.sum(diff * diff)
    counts_ref[...] += jnp.sum(oh, axis=0, keepdims=True)

    @pl.when(i == nblocks - 1)
    def _finalize():
        loss_ref[0, 0] = (1.0 + _COMMITMENT_COST) * loss_acc_ref[0, 0] / (
            n_tok * _EMBEDDING_DIM)
        p = counts_ref[...] / n_tok
        ent = -jnp.sum(p * jnp.log(p + 1e-10))
        perp_ref[0, 0] = jnp.exp(ent)


@functools.partial(jax.jit, static_argnames=())
def _vq(flat_x, cb):
    n_tok = flat_x.shape[0]
    nblocks = n_tok // _TM
    q, loss, perp = pl.pallas_call(
        _vq_kernel,
        grid=(nblocks,),
        in_specs=[
            pl.BlockSpec((_TM, _EMBEDDING_DIM), lambda i: (i, 0)),
            pl.BlockSpec((_NUM_EMBEDDINGS, _EMBEDDING_DIM), lambda i: (0, 0)),
        ],
        out_specs=[
            pl.BlockSpec((_TM, _EMBEDDING_DIM), lambda i: (i, 0)),
            pl.BlockSpec((1, 1), lambda i: (0, 0), memory_space=pltpu.SMEM),
            pl.BlockSpec((1, 1), lambda i: (0, 0), memory_space=pltpu.SMEM),
        ],
        out_shape=[
            jax.ShapeDtypeStruct((n_tok, _EMBEDDING_DIM), jnp.float32),
            jax.ShapeDtypeStruct((1, 1), jnp.float32),
            jax.ShapeDtypeStruct((1, 1), jnp.float32),
        ],
        scratch_shapes=[
            pltpu.VMEM((1, _NUM_EMBEDDINGS), jnp.float32),
            pltpu.SMEM((1, 1), jnp.float32),
        ],
    )(flat_x, cb)
    return q, loss[0, 0], perp[0, 0]


def kernel(inputs, codebook):
    input_shape = inputs.shape
    flat_x = inputs.reshape(-1, _EMBEDDING_DIM).astype(jnp.float32)
    cb = codebook.astype(jnp.float32)
    q, loss, perp = _vq(flat_x, cb)
    quantized_ste = q.reshape(input_shape).astype(inputs.dtype)
    return (quantized_ste, loss, perp)


# speedup vs baseline: 8.1564x; 1.2212x over previous
import functools

import jax
import jax.numpy as jnp
from jax.experimental import pallas as pl
from jax.experimental.pallas import tpu as pltpu

_NUM_EMBEDDINGS = 8192
_EMBEDDING_DIM = 32
_COMMITMENT_COST = 0.25

_TM = 256


def _vq_kernel(x_ref, cb_ref, q_ref, loss_ref, perp_ref, counts_ref, loss_acc_ref):
    i = pl.program_id(0)
    nblocks = pl.num_programs(0)
    n_tok = nblocks * _TM

    x = x_ref[...]
    cb = cb_ref[...]

    x2 = jnp.sum(x * x, axis=1, keepdims=True)
    c2 = jnp.sum(cb * cb, axis=1)[None, :]
    dots = jax.lax.dot_general(
        x, cb,
        dimension_numbers=(((1,), (1,)), ((), ())),
        precision=jax.lax.Precision.DEFAULT,
        preferred_element_type=jnp.float32,
    )
    dist = x2 + c2 - 2.0 * dots

    a = jnp.argmin(dist, axis=1)[:, None]
    iota = jax.lax.broadcasted_iota(jnp.int32, dist.shape, 1)
    oh = (iota == a).astype(jnp.bfloat16)

    q = jax.lax.dot_general(
        oh, cb.astype(jnp.bfloat16),
        dimension_numbers=(((1,), (0,)), ((), ())),
        precision=jax.lax.Precision.DEFAULT,
        preferred_element_type=jnp.float32,
    )
    q_ref[...] = q

    @pl.when(i == 0)
    def _init():
        counts_ref[...] = jnp.zeros_like(counts_ref)
        loss_acc_ref[0, 0] = 0.0

    diff = q - x
    loss_acc_ref[0, 0] += jnp.sum(diff * diff)
    counts_ref[...] += jnp.sum(oh, axis=0, keepdims=True).astype(jnp.float32)

    @pl.when(i == nblocks - 1)
    def _finalize():
        loss_ref[0, 0] = (1.0 + _COMMITMENT_COST) * loss_acc_ref[0, 0] / (
            n_tok * _EMBEDDING_DIM)
        p = counts_ref[...] / n_tok
        ent = -jnp.sum(p * jnp.log(p + 1e-10))
        perp_ref[0, 0] = jnp.exp(ent)


@functools.partial(jax.jit, static_argnames=())
def _vq(flat_x, cb):
    n_tok = flat_x.shape[0]
    nblocks = n_tok // _TM
    q, loss, perp = pl.pallas_call(
        _vq_kernel,
        grid=(nblocks,),
        in_specs=[
            pl.BlockSpec((_TM, _EMBEDDING_DIM), lambda i: (i, 0)),
            pl.BlockSpec((_NUM_EMBEDDINGS, _EMBEDDING_DIM), lambda i: (0, 0)),
        ],
        out_specs=[
            pl.BlockSpec((_TM, _EMBEDDING_DIM), lambda i: (i, 0)),
            pl.BlockSpec((1, 1), lambda i: (0, 0), memory_space=pltpu.SMEM),
            pl.BlockSpec((1, 1), lambda i: (0, 0), memory_space=pltpu.SMEM),
        ],
        out_shape=[
            jax.ShapeDtypeStruct((n_tok, _EMBEDDING_DIM), jnp.float32),
            jax.ShapeDtypeStruct((1, 1), jnp.float32),
            jax.ShapeDtypeStruct((1, 1), jnp.float32),
        ],
        scratch_shapes=[
            pltpu.VMEM((1, _NUM_EMBEDDINGS), jnp.float32),
            pltpu.SMEM((1, 1), jnp.float32),
        ],
    )(flat_x, cb)
    return q, loss[0, 0], perp[0, 0]


def kernel(inputs, codebook):
    input_shape = inputs.shape
    flat_x = inputs.reshape(-1, _EMBEDDING_DIM).astype(jnp.float32)
    cb = codebook.astype(jnp.float32)
    q, loss, perp = _vq(flat_x, cb)
    quantized_ste = q.reshape(input_shape).astype(inputs.dtype)
    return (quantized_ste, loss, perp)


# speedup vs baseline: 8.2943x; 1.0169x over previous
import functools

import jax
import jax.numpy as jnp
from jax.experimental import pallas as pl
from jax.experimental.pallas import tpu as pltpu

_NUM_EMBEDDINGS = 8192
_EMBEDDING_DIM = 32
_COMMITMENT_COST = 0.25

_TM = 512


def _vq_kernel(x_ref, cb_ref, q_ref, loss_ref, perp_ref, counts_ref, loss_acc_ref):
    i = pl.program_id(0)
    nblocks = pl.num_programs(0)
    n_tok = nblocks * _TM

    x = x_ref[...]
    cb = cb_ref[...]

    x2 = jnp.sum(x * x, axis=1, keepdims=True)
    c2 = jnp.sum(cb * cb, axis=1)[None, :]
    dots = jax.lax.dot_general(
        x, cb,
        dimension_numbers=(((1,), (1,)), ((), ())),
        precision=jax.lax.Precision.DEFAULT,
        preferred_element_type=jnp.float32,
    )
    dist = x2 + c2 - 2.0 * dots

    a = jnp.argmin(dist, axis=1)[:, None]
    iota = jax.lax.broadcasted_iota(jnp.int32, dist.shape, 1)
    oh = (iota == a).astype(jnp.bfloat16)

    q = jax.lax.dot_general(
        oh, cb.astype(jnp.bfloat16),
        dimension_numbers=(((1,), (0,)), ((), ())),
        precision=jax.lax.Precision.DEFAULT,
        preferred_element_type=jnp.float32,
    )
    q_ref[...] = q

    @pl.when(i == 0)
    def _init():
        counts_ref[...] = jnp.zeros_like(counts_ref)
        loss_acc_ref[0, 0] = 0.0

    diff = q - x
    loss_acc_ref[0, 0] += jnp.sum(diff * diff)
    counts_ref[...] += jnp.sum(oh, axis=0, keepdims=True).astype(jnp.float32)

    @pl.when(i == nblocks - 1)
    def _finalize():
        loss_ref[0, 0] = (1.0 + _COMMITMENT_COST) * loss_acc_ref[0, 0] / (
            n_tok * _EMBEDDING_DIM)
        p = counts_ref[...] / n_tok
        ent = -jnp.sum(p * jnp.log(p + 1e-10))
        perp_ref[0, 0] = jnp.exp(ent)


@functools.partial(jax.jit, static_argnames=())
def _vq(flat_x, cb):
    n_tok = flat_x.shape[0]
    nblocks = n_tok // _TM
    q, loss, perp = pl.pallas_call(
        _vq_kernel,
        grid=(nblocks,),
        in_specs=[
            pl.BlockSpec((_TM, _EMBEDDING_DIM), lambda i: (i, 0)),
            pl.BlockSpec((_NUM_EMBEDDINGS, _EMBEDDING_DIM), lambda i: (0, 0)),
        ],
        out_specs=[
            pl.BlockSpec((_TM, _EMBEDDING_DIM), lambda i: (i, 0)),
            pl.BlockSpec((1, 1), lambda i: (0, 0), memory_space=pltpu.SMEM),
            pl.BlockSpec((1, 1), lambda i: (0, 0), memory_space=pltpu.SMEM),
        ],
        out_shape=[
            jax.ShapeDtypeStruct((n_tok, _EMBEDDING_DIM), jnp.float32),
            jax.ShapeDtypeStruct((1, 1), jnp.float32),
            jax.ShapeDtypeStruct((1, 1), jnp.float32),
        ],
        scratch_shapes=[
            pltpu.VMEM((1, _NUM_EMBEDDINGS), jnp.float32),
            pltpu.SMEM((1, 1), jnp.float32),
        ],
    )(flat_x, cb)
    return q, loss[0, 0], perp[0, 0]


def kernel(inputs, codebook):
    input_shape = inputs.shape
    flat_x = inputs.reshape(-1, _EMBEDDING_DIM).astype(jnp.float32)
    cb = codebook.astype(jnp.float32)
    q, loss, perp = _vq(flat_x, cb)
    quantized_ste = q.reshape(input_shape).astype(inputs.dtype)
    return (quantized_ste, loss, perp)


# speedup vs baseline: 8.4687x; 1.0210x over previous
import functools

import jax
import jax.numpy as jnp
from jax.experimental import pallas as pl
from jax.experimental.pallas import tpu as pltpu

_NUM_EMBEDDINGS = 8192
_EMBEDDING_DIM = 32
_COMMITMENT_COST = 0.25

_TM = 1024


def _vq_kernel(x_ref, cb_ref, q_ref, loss_ref, perp_ref, counts_ref, loss_acc_ref):
    i = pl.program_id(0)
    nblocks = pl.num_programs(0)
    n_tok = nblocks * _TM

    x = x_ref[...]
    cb = cb_ref[...]

    x2 = jnp.sum(x * x, axis=1, keepdims=True)
    c2 = jnp.sum(cb * cb, axis=1)[None, :]
    dots = jax.lax.dot_general(
        x, cb,
        dimension_numbers=(((1,), (1,)), ((), ())),
        precision=jax.lax.Precision.DEFAULT,
        preferred_element_type=jnp.float32,
    )
    dist = x2 + c2 - 2.0 * dots

    a = jnp.argmin(dist, axis=1)[:, None]
    iota = jax.lax.broadcasted_iota(jnp.int32, dist.shape, 1)
    oh = (iota == a).astype(jnp.bfloat16)

    q = jax.lax.dot_general(
        oh, cb.astype(jnp.bfloat16),
        dimension_numbers=(((1,), (0,)), ((), ())),
        precision=jax.lax.Precision.DEFAULT,
        preferred_element_type=jnp.float32,
    )
    q_ref[...] = q

    @pl.when(i == 0)
    def _init():
        counts_ref[...] = jnp.zeros_like(counts_ref)
        loss_acc_ref[0, 0] = 0.0

    diff = q - x
    loss_acc_ref[0, 0] += jnp.sum(diff * diff)
    counts_ref[...] += jnp.sum(oh, axis=0, keepdims=True).astype(jnp.float32)

    @pl.when(i == nblocks - 1)
    def _finalize():
        loss_ref[0, 0] = (1.0 + _COMMITMENT_COST) * loss_acc_ref[0, 0] / (
            n_tok * _EMBEDDING_DIM)
        p = counts_ref[...] / n_tok
        ent = -jnp.sum(p * jnp.log(p + 1e-10))
        perp_ref[0, 0] = jnp.exp(ent)


@functools.partial(jax.jit, static_argnames=())
def _vq(flat_x, cb):
    n_tok = flat_x.shape[0]
    nblocks = n_tok // _TM
    q, loss, perp = pl.pallas_call(
        _vq_kernel,
        grid=(nblocks,),
        in_specs=[
            pl.BlockSpec((_TM, _EMBEDDING_DIM), lambda i: (i, 0)),
            pl.BlockSpec((_NUM_EMBEDDINGS, _EMBEDDING_DIM), lambda i: (0, 0)),
        ],
        out_specs=[
            pl.BlockSpec((_TM, _EMBEDDING_DIM), lambda i: (i, 0)),
            pl.BlockSpec((1, 1), lambda i: (0, 0), memory_space=pltpu.SMEM),
            pl.BlockSpec((1, 1), lambda i: (0, 0), memory_space=pltpu.SMEM),
        ],
        out_shape=[
            jax.ShapeDtypeStruct((n_tok, _EMBEDDING_DIM), jnp.float32),
            jax.ShapeDtypeStruct((1, 1), jnp.float32),
            jax.ShapeDtypeStruct((1, 1), jnp.float32),
        ],
        scratch_shapes=[
            pltpu.VMEM((1, _NUM_EMBEDDINGS), jnp.float32),
            pltpu.SMEM((1, 1), jnp.float32),
        ],
    )(flat_x, cb)
    return q, loss[0, 0], perp[0, 0]


def kernel(inputs, codebook):
    input_shape = inputs.shape
    flat_x = inputs.reshape(-1, _EMBEDDING_DIM).astype(jnp.float32)
    cb = codebook.astype(jnp.float32)
    q, loss, perp = _vq(flat_x, cb)
    quantized_ste = q.reshape(input_shape).astype(inputs.dtype)
    return (quantized_ste, loss, perp)


# speedup vs baseline: 10.5149x; 1.2416x over previous
import functools

import jax
import jax.numpy as jnp
from jax import lax
from jax.experimental import pallas as pl
from jax.experimental.pallas import tpu as pltpu
from jax.experimental.pallas import tpu_sc as plsc

_K = 8192
_D = 32
_COMMITMENT_COST = 0.25

_TM = 1024
_N_TOK = 4096

_NC = 2
_NS = 16
_NW = _NC * _NS
_TPW = _N_TOK // _NW
_BPW = _K // _NS


def _argmin_kernel(x_ref, cb_ref, idx_ref):
    x = x_ref[...]
    cb = cb_ref[...]
    x2 = jnp.sum(x * x, axis=1, keepdims=True)
    c2 = jnp.sum(cb * cb, axis=1)[None, :]
    dots = jax.lax.dot_general(
        x, cb,
        dimension_numbers=(((1,), (1,)), ((), ())),
        precision=jax.lax.Precision.DEFAULT,
        preferred_element_type=jnp.float32,
    )
    dist = x2 + c2 - 2.0 * dots
    idx_ref[...] = jnp.argmin(dist, axis=1).astype(jnp.int32)[:, None]


@jax.jit
def _tc_argmin(flat_x, cb):
    nblocks = _N_TOK // _TM
    return pl.pallas_call(
        _argmin_kernel,
        grid=(nblocks,),
        in_specs=[
            pl.BlockSpec((_TM, _D), lambda i: (i, 0)),
            pl.BlockSpec((_K, _D), lambda i: (0, 0)),
        ],
        out_specs=pl.BlockSpec((_TM, 1), lambda i: (i, 0)),
        out_shape=jax.ShapeDtypeStruct((_N_TOK, 1), jnp.int32),
    )(flat_x, cb)


def _sc_body(cb_hbm, idx_hbm, q_hbm, counts_hbm,
             idx_v, rows_v, ones_v, zeros_v, shared_counts, sem):
    c = lax.axis_index("c")
    s = lax.axis_index("s")
    wid = c * _NS + s
    base = wid * _TPW

    for i in range(_BPW // 16):
        zeros_v[pl.ds(i * 16, 16)] = jnp.zeros((16,), jnp.float32)
    for i in range(_TPW // 16):
        ones_v[pl.ds(i * 16, 16)] = jnp.ones((16,), jnp.float32)

    pltpu.sync_copy(zeros_v, shared_counts.at[pl.ds(s * _BPW, _BPW)])

    pltpu.sync_copy(idx_hbm.at[pl.ds(base, _TPW)], idx_v)
    pltpu.async_copy(cb_hbm.at[idx_v], rows_v, sem).wait()
    pltpu.sync_copy(rows_v, q_hbm.at[pl.ds(base, _TPW)])

    plsc.subcore_barrier()
    pltpu.sync_copy(ones_v, shared_counts.at[idx_v], add=True)
    plsc.subcore_barrier()

    pltpu.sync_copy(shared_counts.at[pl.ds(s * _BPW, _BPW)],
                    counts_hbm.at[c, pl.ds(s * _BPW, _BPW)])


@jax.jit
def _sc_gather_hist(cb, idx):
    mesh = plsc.VectorSubcoreMesh(
        core_axis_name="c", subcore_axis_name="s",
        num_cores=_NC, num_subcores=_NS)
    f = functools.partial(
        pl.kernel,
        out_type=[
            jax.ShapeDtypeStruct((_N_TOK, _D), jnp.float32),
            jax.ShapeDtypeStruct((_NC, _K), jnp.float32),
        ],
        mesh=mesh,
        scratch_types=[
            pltpu.VMEM((_TPW,), jnp.int32),
            pltpu.VMEM((_TPW, _D), jnp.float32),
            pltpu.VMEM((_TPW,), jnp.float32),
            pltpu.VMEM((_BPW,), jnp.float32),
            pltpu.VMEM_SHARED((_K,), jnp.float32),
            pltpu.SemaphoreType.DMA,
        ],
        compiler_params=pltpu.CompilerParams(use_tc_tiling_on_sc=False),
    )(_sc_body)
    return f(cb, idx)


def _finalize_kernel(x_ref, q_ref, counts_ref, loss_ref, perp_ref):
    x = x_ref[...]
    q = q_ref[...]
    diff = q - x
    loss_ref[0, 0] = (1.0 + _COMMITMENT_COST) * jnp.sum(diff * diff) / (
        _N_TOK * _D)
    counts = counts_ref[0, :] + counts_ref[1, :]
    p = counts / _N_TOK
    ent = -jnp.sum(p * jnp.log(p + 1e-10))
    perp_ref[0, 0] = jnp.exp(ent)


@jax.jit
def _tc_finalize(flat_x, q, counts):
    loss, perp = pl.pallas_call(
        _finalize_kernel,
        in_specs=[
            pl.BlockSpec((_N_TOK, _D), lambda: (0, 0)),
            pl.BlockSpec((_N_TOK, _D), lambda: (0, 0)),
            pl.BlockSpec((_NC, _K), lambda: (0, 0)),
        ],
        out_specs=[
            pl.BlockSpec((1, 1), lambda: (0, 0), memory_space=pltpu.SMEM),
            pl.BlockSpec((1, 1), lambda: (0, 0), memory_space=pltpu.SMEM),
        ],
        out_shape=[
            jax.ShapeDtypeStruct((1, 1), jnp.float32),
            jax.ShapeDtypeStruct((1, 1), jnp.float32),
        ],
    )(flat_x, q, counts)
    return loss[0, 0], perp[0, 0]


def kernel(inputs, codebook):
    input_shape = inputs.shape
    flat_x = inputs.reshape(-1, _D).astype(jnp.float32)
    cb = codebook.astype(jnp.float32)
    idx = _tc_argmin(flat_x, cb).reshape(-1)
    q, counts = _sc_gather_hist(cb, idx)
    loss, perp = _tc_finalize(flat_x, q, counts)
    quantized_ste = q.reshape(input_shape).astype(inputs.dtype)
    return (quantized_ste, loss, perp)
